# baseline (device time: 47529 ns/iter reference)
import jax
import jax.numpy as jnp
from jax import lax
from jax.experimental import pallas as pl
from jax.experimental.pallas import tpu as pltpu

N_DEV = 4
B, Sq, Skv, Dh = 2, 512, 512, 64
H_LOC = 8
D_LOC = H_LOC * Dh
D_MODEL = 768
CH = Sq // N_DEV
NS = 4
SW = D_MODEL // NS
WINDOW = 128

COMM_DT = jnp.bfloat16


def kernel(x, Wq, K_ext, V_ext, Wo):
    def body(x_ref, wq_ref, k_ref, v_ref, wo_ref, out_ref, part_ref,
             rs_stage, rs_recv, ag_stage, ag_recv,
             rs_ssem, rs_rsem, ag_ssem, ag_rsem):
        p = lax.axis_index("i")

        barrier_sem = pltpu.get_barrier_semaphore()
        for d in range(1, N_DEV):
            pl.semaphore_signal(
                barrier_sem, inc=1,
                device_id=((p + d) % N_DEV,),
                device_id_type=pl.DeviceIdType.MESH,
            )

        col0 = p * D_LOC
        wq_loc = wq_ref[:, pl.ds(col0, D_LOC)]
        wo_loc = wo_ref[pl.ds(col0, D_LOC), :]

        qi = lax.broadcasted_iota(jnp.int32, (Sq, Skv), 0)
        ki = lax.broadcasted_iota(jnp.int32, (Sq, Skv), 1)
        mask01 = jnp.where(jnp.abs(qi - ki) <= WINDOW,
                           jnp.float32(1.0), jnp.float32(0.0))

        x2 = x_ref[...].reshape(B * Sq, D_MODEL)
        q_all = jnp.dot(x2, wq_loc,
                        preferred_element_type=jnp.float32) * 0.125

        ctx_rows = []
        for b in range(B):
            ctx_cols = []
            for h in range(H_LOC):
                q = q_all[b * Sq:(b + 1) * Sq, h * Dh:(h + 1) * Dh]
                k = k_ref[b, :, h, :]
                v = v_ref[b, :, h, :]
                s = lax.dot_general(
                    q, k, (((1,), (1,)), ((), ())),
                    preferred_element_type=jnp.float32)
                w = jnp.exp(s) * mask01
                denom = jnp.sum(w, axis=1, keepdims=True)
                ctx_cols.append(
                    jnp.dot(w, v, preferred_element_type=jnp.float32)
                    / denom)
            ctx_rows.append(jnp.concatenate(ctx_cols, axis=1))
        ctx_all = jnp.concatenate(ctx_rows, axis=0)

        def part_chunk(c, sl):
            return part_ref[:, pl.ds((c % N_DEV) * CH, CH),
                            sl * SW:(sl + 1) * SW]

        def put_out(c, sl, val):
            out_ref[:, pl.ds((c % N_DEV) * CH, CH),
                    sl * SW:(sl + 1) * SW] = val.astype(jnp.float32)

        def start(src, dst, ssem, rsem, slot, dest):
            rdma = pltpu.make_async_remote_copy(
                src_ref=src.at[slot], dst_ref=dst.at[slot],
                send_sem=ssem.at[slot], recv_sem=rsem.at[slot],
                device_id=(dest,), device_id_type=pl.DeviceIdType.MESH,
            )
            rdma.start()
            return rdma

        def project(sl):
            part_ref[:, :, sl * SW:(sl + 1) * SW] = jnp.dot(
                ctx_all, wo_loc[:, sl * SW:(sl + 1) * SW],
                preferred_element_type=jnp.float32).reshape(B, Sq, SW)

        def stage_rs(sl):
            for d in range(1, N_DEV):
                rs_stage[sl * 3 + d - 1] = (
                    part_chunk(p + d, sl).astype(COMM_DT))

        def start_rs(sl):
            return {d: start(rs_stage, rs_recv, rs_ssem, rs_rsem,
                             sl * 3 + d - 1, (p + d) % N_DEV)
                    for d in (2, 1, 3)}

        def start_ag(sl, d):
            rdma = pltpu.make_async_remote_copy(
                src_ref=ag_stage.at[sl],
                dst_ref=ag_recv.at[sl * 3 + d - 1],
                send_sem=ag_ssem.at[sl * 3 + d - 1],
                recv_sem=ag_rsem.at[sl * 3 + d - 1],
                device_id=((p + d) % N_DEV,),
                device_id_type=pl.DeviceIdType.MESH,
            )
            rdma.start()
            return rdma

        rs = {}
        for sl in range(NS):
            project(sl)
            stage_rs(sl)
            if sl == 0:
                pl.semaphore_wait(barrier_sem, N_DEV - 1)
            rs[sl] = start_rs(sl)

        ag = {}
        accs = {}
        for sl in range(NS):
            acc = part_chunk(p, sl)
            for d in (1, 3, 2):
                rs[sl][d].wait_recv()
                acc = acc + rs_recv[sl * 3 + d - 1].astype(jnp.float32)
            accs[sl] = acc
            ag_stage[sl] = acc.astype(COMM_DT)
            ag[sl] = {d: start_ag(sl, d) for d in (2, 1, 3)}

        for sl in range(NS):
            put_out(p, sl, accs[sl])
        for sl in range(NS):
            for d in (1, 3, 2):
                ag[sl][d].wait_recv()
                put_out(p - d, sl, ag_recv[sl * 3 + d - 1])

        for sl in range(NS):
            for d in (1, 2, 3):
                rs[sl][d].wait_send()
                ag[sl][d].wait_send()

    chunk = (B, CH, SW)
    n_slot = NS * (N_DEV - 1)
    return pl.pallas_call(
        body,
        out_shape=jax.ShapeDtypeStruct((B, Sq, D_MODEL), jnp.float32),
        in_specs=[pl.BlockSpec(memory_space=pltpu.VMEM)] * 5,
        out_specs=pl.BlockSpec(memory_space=pltpu.VMEM),
        scratch_shapes=[
            pltpu.VMEM((B, Sq, D_MODEL), jnp.float32),
            pltpu.VMEM((n_slot,) + chunk, COMM_DT),
            pltpu.VMEM((n_slot,) + chunk, COMM_DT),
            pltpu.VMEM((NS,) + chunk, COMM_DT),
            pltpu.VMEM((n_slot,) + chunk, COMM_DT),
            pltpu.SemaphoreType.DMA((n_slot,)),
            pltpu.SemaphoreType.DMA((n_slot,)),
            pltpu.SemaphoreType.DMA((n_slot,)),
            pltpu.SemaphoreType.DMA((n_slot,)),
        ],
        compiler_params=pltpu.CompilerParams(collective_id=0),
    )(x, Wq, K_ext, V_ext, Wo)


# device time: 42355 ns/iter; 1.1222x vs baseline; 1.1222x over previous
import jax
import jax.numpy as jnp
from jax import lax
from jax.experimental import pallas as pl
from jax.experimental.pallas import tpu as pltpu

N_DEV = 4
B, Sq, Skv, Dh = 2, 512, 512, 64
H_LOC = 8
D_LOC = H_LOC * Dh
D_MODEL = 768
CH = Sq // N_DEV
NS = 3
SW = D_MODEL // NS
WINDOW = 128

COMM_DT = jnp.bfloat16


def kernel(x, Wq, K_ext, V_ext, Wo):
    def body(x_ref, wq_ref, k_ref, v_ref, wo_ref, out_ref, part_ref,
             rs_stage, rs_recv, ag_stage, ag_recv,
             rs_ssem, rs_rsem, ag_ssem, ag_rsem):
        p = lax.axis_index("i")

        barrier_sem = pltpu.get_barrier_semaphore()
        for d in range(1, N_DEV):
            pl.semaphore_signal(
                barrier_sem, inc=1,
                device_id=((p + d) % N_DEV,),
                device_id_type=pl.DeviceIdType.MESH,
            )

        col0 = p * D_LOC
        wq_loc = wq_ref[:, pl.ds(col0, D_LOC)]
        wo_loc = wo_ref[pl.ds(col0, D_LOC), :]

        qi = lax.broadcasted_iota(jnp.int32, (Sq, Skv), 0)
        ki = lax.broadcasted_iota(jnp.int32, (Sq, Skv), 1)
        mask01 = jnp.where(jnp.abs(qi - ki) <= WINDOW,
                           jnp.float32(1.0), jnp.float32(0.0))

        x2 = x_ref[...].reshape(B * Sq, D_MODEL)
        q_all = jnp.dot(x2, wq_loc,
                        preferred_element_type=jnp.float32) * 0.125

        ctx_rows = []
        for b in range(B):
            ctx_cols = []
            for h in range(H_LOC):
                q = q_all[b * Sq:(b + 1) * Sq, h * Dh:(h + 1) * Dh]
                k = k_ref[b, :, h, :]
                v = v_ref[b, :, h, :]
                s = lax.dot_general(
                    q, k, (((1,), (1,)), ((), ())),
                    preferred_element_type=jnp.float32)
                w = jnp.exp(s) * mask01
                denom = jnp.sum(w, axis=1, keepdims=True)
                ctx_cols.append(
                    jnp.dot(w, v, preferred_element_type=jnp.float32)
                    / denom)
            ctx_rows.append(jnp.concatenate(ctx_cols, axis=1))
        ctx_all = jnp.concatenate(ctx_rows, axis=0)

        def part_chunk(c, sl):
            return part_ref[:, pl.ds((c % N_DEV) * CH, CH),
                            sl * SW:(sl + 1) * SW]

        def put_out(c, sl, val):
            out_ref[:, pl.ds((c % N_DEV) * CH, CH),
                    sl * SW:(sl + 1) * SW] = val.astype(jnp.float32)

        def start(src, dst, ssem, rsem, slot, dest):
            rdma = pltpu.make_async_remote_copy(
                src_ref=src.at[slot], dst_ref=dst.at[slot],
                send_sem=ssem.at[slot], recv_sem=rsem.at[slot],
                device_id=(dest,), device_id_type=pl.DeviceIdType.MESH,
            )
            rdma.start()
            return rdma

        def project(sl):
            part_ref[:, :, sl * SW:(sl + 1) * SW] = jnp.dot(
                ctx_all, wo_loc[:, sl * SW:(sl + 1) * SW],
                preferred_element_type=jnp.float32).reshape(B, Sq, SW)

        def stage_rs(sl):
            for d in range(1, N_DEV):
                rs_stage[sl * 3 + d - 1] = (
                    part_chunk(p + d, sl).astype(COMM_DT))

        def start_rs(sl):
            return {d: start(rs_stage, rs_recv, rs_ssem, rs_rsem,
                             sl * 3 + d - 1, (p + d) % N_DEV)
                    for d in (2, 1, 3)}

        def start_ag(sl, d):
            rdma = pltpu.make_async_remote_copy(
                src_ref=ag_stage.at[sl],
                dst_ref=ag_recv.at[sl * 3 + d - 1],
                send_sem=ag_ssem.at[sl * 3 + d - 1],
                recv_sem=ag_rsem.at[sl * 3 + d - 1],
                device_id=((p + d) % N_DEV,),
                device_id_type=pl.DeviceIdType.MESH,
            )
            rdma.start()
            return rdma

        rs = {}
        for sl in range(NS):
            project(sl)
            stage_rs(sl)
            if sl == 0:
                pl.semaphore_wait(barrier_sem, N_DEV - 1)
            rs[sl] = start_rs(sl)

        ag = {}
        accs = {}
        for sl in range(NS):
            acc = part_chunk(p, sl)
            for d in (1, 3, 2):
                rs[sl][d].wait_recv()
                acc = acc + rs_recv[sl * 3 + d - 1].astype(jnp.float32)
            accs[sl] = acc
            ag_stage[sl] = acc.astype(COMM_DT)
            ag[sl] = {d: start_ag(sl, d) for d in (2, 1, 3)}

        for sl in range(NS):
            put_out(p, sl, accs[sl])
        for sl in range(NS):
            for d in (1, 3, 2):
                ag[sl][d].wait_recv()
                put_out(p - d, sl, ag_recv[sl * 3 + d - 1])

        for sl in range(NS):
            for d in (1, 2, 3):
                rs[sl][d].wait_send()
                ag[sl][d].wait_send()

    chunk = (B, CH, SW)
    n_slot = NS * (N_DEV - 1)
    return pl.pallas_call(
        body,
        out_shape=jax.ShapeDtypeStruct((B, Sq, D_MODEL), jnp.float32),
        in_specs=[pl.BlockSpec(memory_space=pltpu.VMEM)] * 5,
        out_specs=pl.BlockSpec(memory_space=pltpu.VMEM),
        scratch_shapes=[
            pltpu.VMEM((B, Sq, D_MODEL), jnp.float32),
            pltpu.VMEM((n_slot,) + chunk, COMM_DT),
            pltpu.VMEM((n_slot,) + chunk, COMM_DT),
            pltpu.VMEM((NS,) + chunk, COMM_DT),
            pltpu.VMEM((n_slot,) + chunk, COMM_DT),
            pltpu.SemaphoreType.DMA((n_slot,)),
            pltpu.SemaphoreType.DMA((n_slot,)),
            pltpu.SemaphoreType.DMA((n_slot,)),
            pltpu.SemaphoreType.DMA((n_slot,)),
        ],
        compiler_params=pltpu.CompilerParams(collective_id=0),
    )(x, Wq, K_ext, V_ext, Wo)


# device time: 41195 ns/iter; 1.1538x vs baseline; 1.0282x over previous
import jax
import jax.numpy as jnp
from jax import lax
from jax.experimental import pallas as pl
from jax.experimental.pallas import tpu as pltpu

N_DEV = 4
B, Sq, Skv, Dh = 2, 512, 512, 64
H_LOC = 8
D_LOC = H_LOC * Dh
D_MODEL = 768
CH = Sq // N_DEV
HALF = D_MODEL // 2
WINDOW = 128

COMM_DT = jnp.bfloat16


def kernel(x, Wq, K_ext, V_ext, Wo):
    def body(x_ref, wq_ref, k_ref, v_ref, wo_ref, out_ref, part_ref,
             rs_stage, rs_recv, ag_stage, ag_recv,
             rs_ssem, rs_rsem, ag_ssem, ag_rsem):
        p = lax.axis_index("i")

        barrier_sem = pltpu.get_barrier_semaphore()
        for d in range(1, N_DEV):
            pl.semaphore_signal(
                barrier_sem, inc=1,
                device_id=((p + d) % N_DEV,),
                device_id_type=pl.DeviceIdType.MESH,
            )

        col0 = p * D_LOC
        wq_loc = wq_ref[:, pl.ds(col0, D_LOC)]
        wo_loc = wo_ref[pl.ds(col0, D_LOC), :]

        qi = lax.broadcasted_iota(jnp.int32, (Sq, Skv), 0)
        ki = lax.broadcasted_iota(jnp.int32, (Sq, Skv), 1)
        mask01 = jnp.where(jnp.abs(qi - ki) <= WINDOW,
                           jnp.float32(1.0), jnp.float32(0.0))

        x2 = x_ref[...].reshape(B * Sq, D_MODEL)
        q_all = jnp.dot(x2, wq_loc,
                        preferred_element_type=jnp.float32) * 0.125

        ctx_rows = []
        for b in range(B):
            ctx_cols = []
            for h in range(H_LOC):
                q = q_all[b * Sq:(b + 1) * Sq, h * Dh:(h + 1) * Dh]
                k = k_ref[b, :, h, :]
                v = v_ref[b, :, h, :]
                s = lax.dot_general(
                    q, k, (((1,), (1,)), ((), ())),
                    preferred_element_type=jnp.float32)
                w = jnp.exp(s) * mask01
                denom = jnp.sum(w, axis=1, keepdims=True)
                ctx_cols.append(
                    jnp.dot(w, v, preferred_element_type=jnp.float32)
                    / denom)
            ctx_rows.append(jnp.concatenate(ctx_cols, axis=1))
        ctx_all = jnp.concatenate(ctx_rows, axis=0)

        def part_chunk(c, half):
            return part_ref[:, pl.ds((c % N_DEV) * CH, CH),
                            half * HALF:(half + 1) * HALF]

        def put_out(c, half, val):
            out_ref[:, pl.ds((c % N_DEV) * CH, CH),
                    half * HALF:(half + 1) * HALF] = val.astype(jnp.float32)

        def start(src, dst, ssem, rsem, slot, dest):
            rdma = pltpu.make_async_remote_copy(
                src_ref=src.at[slot], dst_ref=dst.at[slot],
                send_sem=ssem.at[slot], recv_sem=rsem.at[slot],
                device_id=(dest,), device_id_type=pl.DeviceIdType.MESH,
            )
            rdma.start()
            return rdma

        def project(half):
            part_ref[:, :, half * HALF:(half + 1) * HALF] = jnp.dot(
                ctx_all, wo_loc[:, half * HALF:(half + 1) * HALF],
                preferred_element_type=jnp.float32).reshape(B, Sq, HALF)

        def stage_rs(half):
            for d in range(1, N_DEV):
                rs_stage[half * 3 + d - 1] = (
                    part_chunk(p + d, half).astype(COMM_DT))

        def start_rs(half):
            return {d: start(rs_stage, rs_recv, rs_ssem, rs_rsem,
                             half * 3 + d - 1, (p + d) % N_DEV)
                    for d in (2, 1, 3)}

        project(0)
        stage_rs(0)
        pl.semaphore_wait(barrier_sem, N_DEV - 1)
        rs0 = start_rs(0)

        project(1)
        stage_rs(1)
        rs1 = start_rs(1)

        rs = {0: rs0, 1: rs1}
        ag = {}
        accs = {}
        for half in (0, 1):
            acc = part_chunk(p, half)
            for d in (1, 3, 2):
                rs[half][d].wait_recv()
                acc = acc + rs_recv[half * 3 + d - 1].astype(jnp.float32)
            accs[half] = acc
            ag_stage[half] = acc.astype(COMM_DT)

            def start_ag(h, d):
                rdma = pltpu.make_async_remote_copy(
                    src_ref=ag_stage.at[h],
                    dst_ref=ag_recv.at[h * 3 + d - 1],
                    send_sem=ag_ssem.at[h * 3 + d - 1],
                    recv_sem=ag_rsem.at[h * 3 + d - 1],
                    device_id=((p + d) % N_DEV,),
                    device_id_type=pl.DeviceIdType.MESH,
                )
                rdma.start()
                return rdma

            ag[half] = {d: start_ag(half, d) for d in (2, 1, 3)}

        put_out(p, 0, accs[0])
        put_out(p, 1, accs[1])
        for half in (0, 1):
            for d in (1, 3, 2):
                ag[half][d].wait_recv()
                put_out(p - d, half, ag_recv[half * 3 + d - 1])

        for half in (0, 1):
            for d in (1, 2, 3):
                rs[half][d].wait_send()
                ag[half][d].wait_send()

    chunk = (B, CH, HALF)
    n_slot = 2 * (N_DEV - 1)
    return pl.pallas_call(
        body,
        out_shape=jax.ShapeDtypeStruct((B, Sq, D_MODEL), jnp.float32),
        in_specs=[pl.BlockSpec(memory_space=pltpu.VMEM)] * 5,
        out_specs=pl.BlockSpec(memory_space=pltpu.VMEM),
        scratch_shapes=[
            pltpu.VMEM((B, Sq, D_MODEL), jnp.float32),
            pltpu.VMEM((n_slot,) + chunk, COMM_DT),
            pltpu.VMEM((n_slot,) + chunk, COMM_DT),
            pltpu.VMEM((2,) + chunk, COMM_DT),
            pltpu.VMEM((n_slot,) + chunk, COMM_DT),
            pltpu.SemaphoreType.DMA((n_slot,)),
            pltpu.SemaphoreType.DMA((n_slot,)),
            pltpu.SemaphoreType.DMA((n_slot,)),
            pltpu.SemaphoreType.DMA((n_slot,)),
        ],
        compiler_params=pltpu.CompilerParams(collective_id=0),
    )(x, Wq, K_ext, V_ext, Wo)
